# baseline (speedup 1.0000x reference)
import functools

import jax
import jax.numpy as jnp
from jax import lax
from jax.experimental import pallas as pl
from jax.experimental.pallas import tpu as pltpu
from jax.experimental.pallas import tpu_sc as plsc

_NC = 2
_NS = 16
_NW = _NC * _NS

_KMAX = 128
_NBUF = 4


@functools.lru_cache(maxsize=None)
def _build(B, H, D):
    mesh = plsc.VectorSubcoreMesh(core_axis_name="c", subcore_axis_name="s")
    C = B // _NW

    segs = []
    o = 0
    while o < H:
        w = min(_KMAX, H - o)
        segs.append((o, w))
        o += w

    scratch = [
        pltpu.VMEM((C, H), jnp.int32),
        pltpu.VMEM((_NBUF, H, D), jnp.float32),
    ]
    scratch += [pltpu.SemaphoreType.DMA] * (2 * _NBUF)

    @functools.partial(
        pl.kernel,
        mesh=mesh,
        out_type=jax.ShapeDtypeStruct((B, H, D), jnp.float32),
        scratch_types=scratch,
        compiler_params=pltpu.CompilerParams(use_tc_tiling_on_sc=False),
    )
    def emb(idx_hbm, tab_hbm, out_hbm, idx_v, rows_v, *sems):
        gsem = sems[:_NBUF]
        ssem = sems[_NBUF:]
        wid = lax.axis_index("s") * _NC + lax.axis_index("c")
        base = wid * C

        pltpu.sync_copy(idx_hbm.at[pl.ds(base, C)], idx_v)

        def g_start(j, b):
            for (o, w) in segs:
                pltpu.make_async_copy(
                    tab_hbm.at[idx_v.at[j, pl.ds(o, w)]],
                    rows_v.at[b, pl.ds(o, w)], gsem[b]).start()

        def g_wait(j, b):
            for (o, w) in segs:
                pltpu.make_async_copy(
                    tab_hbm.at[idx_v.at[j, pl.ds(o, w)]],
                    rows_v.at[b, pl.ds(o, w)], gsem[b]).wait()

        def s_copy(j, b):
            return pltpu.make_async_copy(
                rows_v.at[b], out_hbm.at[base + j], ssem[b])

        G = C // _NBUF

        for b in range(_NBUF):
            g_start(b, b)

        for b in range(_NBUF):
            g_wait(b, b)
            s_copy(b, b).start()
            if b >= 1:
                s_copy(b - 1, b - 1).wait()
                g_start(b + _NBUF - 1, b - 1)

        def group(g, carry):
            for b in range(_NBUF):
                j = g * _NBUF + b
                g_wait(j, b)
                s_copy(j, b).start()
                bp = (b - 1) % _NBUF
                s_copy(j - 1, bp).wait()
                g_start(j + _NBUF - 1, bp)
            return carry

        lax.fori_loop(1, G - 1, group, 0)

        for b in range(_NBUF):
            j = (G - 1) * _NBUF + b
            g_wait(j, b)
            s_copy(j, b).start()
            bp = (b - 1) % _NBUF
            if b == 0:
                s_copy(j - 1, bp).wait()
                g_start(j + _NBUF - 1, bp)
            else:
                s_copy(j - 1, bp).wait()
        s_copy(C - 1, _NBUF - 1).wait()

    return emb


def kernel(input, weight):
    B, H = input.shape
    V, D = weight.shape
    assert B % _NW == 0 and (B // _NW) % _NBUF == 0
    idx = input.astype(jnp.int32)
    return _build(B, H, D)(idx, weight)
